# baseline (speedup 1.0000x reference)
import functools

import jax
import jax.numpy as jnp
from jax import lax
from jax.experimental import pallas as pl
from jax.experimental.pallas import tpu as pltpu
from jax.experimental.pallas import tpu_sc as plsc

_NC = 2
_NS = 16
_NW = _NC * _NS
_L = 16


@functools.lru_cache(maxsize=None)
def _sc_indicator(E: int, N: int, nc: int = _NC):
    nw = nc * _NS
    assert E % 128 == 0 and N % _L == 0
    chunk = ((E + nw - 1) // nw + 127) // 128 * 128
    assert chunk % _L == 0
    mesh = plsc.VectorSubcoreMesh(core_axis_name="c", subcore_axis_name="s",
                                  num_cores=nc)

    nchunk = ((N + _NS - 1) // _NS + 127) // 128 * 128
    n_pad = _NS * nchunk

    @functools.partial(
        pl.kernel,
        mesh=mesh,
        compiler_params=pltpu.CompilerParams(needs_layout_passes=False),
        out_type=[jax.ShapeDtypeStruct((n_pad,), jnp.float32)
                  for _ in range(nc)],
        scratch_types=[
            pltpu.VMEM((2, chunk), jnp.int32),
            pltpu.VMEM((n_pad,), jnp.float32),
            pltpu.VMEM((_NS, nchunk), jnp.float32),
            pltpu.VMEM((nchunk,), jnp.float32),
            pltpu.VMEM_SHARED((_NS, n_pad), jnp.float32),
        ],
    )
    def body(ei_hbm, zeros_hbm, *rest):
        out_hbms = rest[:nc]
        idx_v, ind_v, blk_v, acc_v, shared = rest[nc:]
        cid = lax.axis_index("c")
        sid = lax.axis_index("s")
        wid = sid * nc + cid
        ones = jnp.ones((_L,), jnp.float32)

        start = jnp.minimum(wid * chunk, E - chunk)

        pltpu.sync_copy(zeros_hbm, ind_v)
        pltpu.sync_copy(ei_hbm.at[:, pl.ds(start, chunk)], idx_v)

        def scat_body(g, carry):
            idx = idx_v[0, pl.ds(pl.multiple_of(g * _L, _L), _L)]
            plsc.store_scatter(ind_v, [idx], ones)
            return carry

        lax.fori_loop(0, chunk // _L, scat_body, 0, unroll=8)

        pltpu.sync_copy(ind_v, shared.at[sid])
        plsc.subcore_barrier()
        cs = sid * nchunk
        pltpu.sync_copy(shared.at[:, pl.ds(cs, nchunk)], blk_v)

        def merge_body(g, carry):
            o = pl.multiple_of(g * _L, _L)
            acc = blk_v[0, pl.ds(o, _L)]
            for t in range(1, _NS):
                acc = jnp.maximum(acc, blk_v[t, pl.ds(o, _L)])
            acc_v[pl.ds(o, _L)] = acc
            return carry

        lax.fori_loop(0, nchunk // _L, merge_body, 0, unroll=2)

        for c in range(nc):
            @pl.when(cid == c)
            def _():
                pltpu.sync_copy(acc_v, out_hbms[c].at[pl.ds(cs, nchunk)])

    return body


_INV_SQRT2 = 0.7071067811865476


def _tc_dense_body(x_ref, w_ref, w1_ref, b1_ref, w2_ref, b2_ref, h_ref,
                   pw_ref):
    h = jnp.dot(x_ref[...], w_ref[...], preferred_element_type=jnp.float32)
    t = jnp.dot(h, w1_ref[...], preferred_element_type=jnp.float32)
    t = t + b1_ref[...]
    g = 0.5 * t * (1.0 + lax.erf(t * _INV_SQRT2))
    p = jnp.sum(g * w2_ref[...], axis=1, keepdims=True) + b2_ref[...]
    h_ref[...] = h
    pw_ref[...] = 1.0 / (1.0 + jnp.exp(-p))


@functools.lru_cache(maxsize=None)
def _tc_dense(N: int, IN: int, OUT: int, HID: int, R: int):
    assert N % R == 0
    grid = (N // R,)
    return pl.pallas_call(
        _tc_dense_body,
        grid=grid,
        in_specs=[
            pl.BlockSpec((R, IN), lambda i: (i, 0)),
            pl.BlockSpec((IN, OUT), lambda i: (0, 0)),
            pl.BlockSpec((OUT, HID), lambda i: (0, 0)),
            pl.BlockSpec((1, HID), lambda i: (0, 0)),
            pl.BlockSpec((1, HID), lambda i: (0, 0)),
            pl.BlockSpec((1, 1), lambda i: (0, 0)),
        ],
        out_specs=[
            pl.BlockSpec((R, OUT), lambda i: (i, 0)),
            pl.BlockSpec((R, 1), lambda i: (i, 0)),
        ],
        out_shape=[
            jax.ShapeDtypeStruct((N, OUT), jnp.float32),
            jax.ShapeDtypeStruct((N, 1), jnp.float32),
        ],
        compiler_params=pltpu.CompilerParams(
            dimension_semantics=("parallel",)),
    )


def _tc_scale_body(h_ref, pw_ref, bias_ref, inda_ref, indb_ref, out_ref):
    ind = jnp.maximum(inda_ref[...], indb_ref[...])
    scale = jnp.where(ind > 0.0, 1.0 + pw_ref[...], 0.0)
    out_ref[...] = h_ref[...] * scale + bias_ref[...]


@functools.lru_cache(maxsize=None)
def _tc_scale(N: int, OUT: int, R: int):
    assert N % R == 0
    grid = (N // R,)
    return pl.pallas_call(
        _tc_scale_body,
        grid=grid,
        in_specs=[
            pl.BlockSpec((R, OUT), lambda i: (i, 0)),
            pl.BlockSpec((R, 1), lambda i: (i, 0)),
            pl.BlockSpec((1, OUT), lambda i: (0, 0)),
            pl.BlockSpec((R, 1), lambda i: (i, 0)),
            pl.BlockSpec((R, 1), lambda i: (i, 0)),
        ],
        out_specs=pl.BlockSpec((R, OUT), lambda i: (i, 0)),
        out_shape=jax.ShapeDtypeStruct((N, OUT), jnp.float32),
        compiler_params=pltpu.CompilerParams(
            dimension_semantics=("parallel",)),
    )


def kernel(x, edge_index, W, att, bias, pd_w1, pd_b1, pd_w2, pd_b2):
    del att
    N, IN = x.shape
    OUT = W.shape[1]
    HID = pd_w1.shape[1]
    E = edge_index.shape[1]

    n_pad = _NS * (((N + _NS - 1) // _NS + 127) // 128 * 128)
    ind_a, ind_b = _sc_indicator(E, N)(edge_index,
                                       jnp.zeros((n_pad,), jnp.float32))

    h, pw = _tc_dense(N, IN, OUT, HID, 1000)(
        x, W, pd_w1,
        pd_b1.reshape(1, HID),
        pd_w2.reshape(1, HID),
        pd_b2.reshape(1, 1),
    )
    return _tc_scale(N, OUT, 1000)(h, pw, bias.reshape(1, OUT),
                                   ind_a.reshape(n_pad, 1),
                                   ind_b.reshape(n_pad, 1))


# speedup vs baseline: 1.1956x; 1.1956x over previous
import functools

import jax
import jax.numpy as jnp
from jax import lax
from jax.experimental import pallas as pl
from jax.experimental.pallas import tpu as pltpu
from jax.experimental.pallas import tpu_sc as plsc

_NC = 2
_NS = 16
_NW = _NC * _NS
_L = 16


@functools.lru_cache(maxsize=None)
def _sc_indicator(E: int, N: int, nc: int = _NC):
    nw = nc * _NS
    assert E % 128 == 0 and N % _L == 0
    chunk = ((E + nw - 1) // nw + 127) // 128 * 128
    assert chunk % _L == 0
    mesh = plsc.VectorSubcoreMesh(core_axis_name="c", subcore_axis_name="s",
                                  num_cores=nc)

    nchunk = ((N + _NS - 1) // _NS + 127) // 128 * 128
    n_pad = _NS * nchunk

    @functools.partial(
        pl.kernel,
        mesh=mesh,
        compiler_params=pltpu.CompilerParams(needs_layout_passes=False),
        out_type=jax.ShapeDtypeStruct((nc, n_pad), jnp.float32),
        scratch_types=[
            pltpu.VMEM((2, chunk), jnp.int32),
            pltpu.VMEM((n_pad,), jnp.float32),
            pltpu.VMEM((_NS, nchunk), jnp.float32),
            pltpu.VMEM((nchunk,), jnp.float32),
            pltpu.VMEM_SHARED((_NS, n_pad), jnp.float32),
        ],
    )
    def body(ei_hbm, zeros_hbm, out_hbm, idx_v, ind_v, blk_v, acc_v, shared):
        cid = lax.axis_index("c")
        sid = lax.axis_index("s")
        wid = sid * nc + cid
        ones = jnp.ones((_L,), jnp.float32)

        start = jnp.minimum(wid * chunk, E - chunk)

        pltpu.sync_copy(zeros_hbm, ind_v)
        pltpu.sync_copy(ei_hbm.at[:, pl.ds(start, chunk)], idx_v)

        def scat_body(g, carry):
            idx = idx_v[0, pl.ds(pl.multiple_of(g * _L, _L), _L)]
            plsc.store_scatter(ind_v, [idx], ones)
            return carry

        lax.fori_loop(0, chunk // _L, scat_body, 0, unroll=8)

        pltpu.sync_copy(ind_v, shared.at[sid])
        plsc.subcore_barrier()
        cs = sid * nchunk
        pltpu.sync_copy(shared.at[:, pl.ds(cs, nchunk)], blk_v)

        def merge_body(g, carry):
            o = pl.multiple_of(g * _L, _L)
            acc = blk_v[0, pl.ds(o, _L)]
            for t in range(1, _NS):
                acc = jnp.maximum(acc, blk_v[t, pl.ds(o, _L)])
            acc_v[pl.ds(o, _L)] = acc
            return carry

        lax.fori_loop(0, nchunk // _L, merge_body, 0, unroll=2)

        pltpu.sync_copy(acc_v, out_hbm.at[cid, pl.ds(cs, nchunk)])

    return body


_INV_SQRT2 = 0.7071067811865476


def _tc_dense_body(x_ref, w_ref, w1_ref, b1_ref, w2_ref, b2_ref, h_ref,
                   pw_ref):
    h = jnp.dot(x_ref[...], w_ref[...], preferred_element_type=jnp.float32)
    t = jnp.dot(h, w1_ref[...], preferred_element_type=jnp.float32)
    t = t + b1_ref[...]
    g = 0.5 * t * (1.0 + lax.erf(t * _INV_SQRT2))
    p = jnp.sum(g * w2_ref[...], axis=1, keepdims=True) + b2_ref[...]
    h_ref[...] = h
    pw_ref[...] = 1.0 / (1.0 + jnp.exp(-p))


@functools.lru_cache(maxsize=None)
def _tc_dense(N: int, IN: int, OUT: int, HID: int, R: int):
    assert N % R == 0
    grid = (N // R,)
    return pl.pallas_call(
        _tc_dense_body,
        grid=grid,
        in_specs=[
            pl.BlockSpec((R, IN), lambda i: (i, 0)),
            pl.BlockSpec((IN, OUT), lambda i: (0, 0)),
            pl.BlockSpec((OUT, HID), lambda i: (0, 0)),
            pl.BlockSpec((1, HID), lambda i: (0, 0)),
            pl.BlockSpec((1, HID), lambda i: (0, 0)),
            pl.BlockSpec((1, 1), lambda i: (0, 0)),
        ],
        out_specs=[
            pl.BlockSpec((R, OUT), lambda i: (i, 0)),
            pl.BlockSpec((R, 1), lambda i: (i, 0)),
        ],
        out_shape=[
            jax.ShapeDtypeStruct((N, OUT), jnp.float32),
            jax.ShapeDtypeStruct((N, 1), jnp.float32),
        ],
        compiler_params=pltpu.CompilerParams(
            dimension_semantics=("parallel",)),
    )


def _tc_scale_body(h_ref, pw_ref, bias_ref, ind_ref, out_ref):
    ind = jnp.max(ind_ref[...], axis=0, keepdims=True)
    indT = jnp.transpose(ind)
    scale = jnp.where(indT > 0.0, 1.0 + pw_ref[...], 0.0)
    out_ref[...] = h_ref[...] * scale + bias_ref[...]


@functools.lru_cache(maxsize=None)
def _tc_scale(N: int, OUT: int, nc: int, R: int):
    grid = ((N + R - 1) // R,)
    return pl.pallas_call(
        _tc_scale_body,
        grid=grid,
        in_specs=[
            pl.BlockSpec((R, OUT), lambda i: (i, 0)),
            pl.BlockSpec((R, 1), lambda i: (i, 0)),
            pl.BlockSpec((1, OUT), lambda i: (0, 0)),
            pl.BlockSpec((nc, R), lambda i: (0, i)),
        ],
        out_specs=pl.BlockSpec((R, OUT), lambda i: (i, 0)),
        out_shape=jax.ShapeDtypeStruct((N, OUT), jnp.float32),
        compiler_params=pltpu.CompilerParams(
            dimension_semantics=("parallel",)),
    )


def kernel(x, edge_index, W, att, bias, pd_w1, pd_b1, pd_w2, pd_b2):
    del att
    N, IN = x.shape
    OUT = W.shape[1]
    HID = pd_w1.shape[1]
    E = edge_index.shape[1]

    n_pad = _NS * (((N + _NS - 1) // _NS + 127) // 128 * 128)
    ind2 = _sc_indicator(E, N)(edge_index, jnp.zeros((n_pad,), jnp.float32))

    h, pw = _tc_dense(N, IN, OUT, HID, 1000)(
        x, W, pd_w1,
        pd_b1.reshape(1, HID),
        pd_w2.reshape(1, HID),
        pd_b2.reshape(1, 1),
    )
    return _tc_scale(N, OUT, _NC, 1024)(h, pw, bias.reshape(1, OUT), ind2)


# speedup vs baseline: 1.2280x; 1.0271x over previous
import functools

import jax
import jax.numpy as jnp
from jax import lax
from jax.experimental import pallas as pl
from jax.experimental.pallas import tpu as pltpu
from jax.experimental.pallas import tpu_sc as plsc

_NC = 2
_NS = 16
_NW = _NC * _NS
_L = 16


@functools.lru_cache(maxsize=None)
def _sc_indicator(E: int, N: int, nc: int = _NC):
    nw = nc * _NS
    assert E % 128 == 0 and N % _L == 0
    chunk = ((E + nw - 1) // nw + 127) // 128 * 128
    assert chunk % _L == 0
    mesh = plsc.VectorSubcoreMesh(core_axis_name="c", subcore_axis_name="s",
                                  num_cores=nc)

    nchunk = ((N + _NS - 1) // _NS + 127) // 128 * 128
    n_pad = _NS * nchunk

    @functools.partial(
        pl.kernel,
        mesh=mesh,
        compiler_params=pltpu.CompilerParams(needs_layout_passes=False),
        out_type=jax.ShapeDtypeStruct((nc, n_pad), jnp.float32),
        scratch_types=[
            pltpu.VMEM((2, chunk), jnp.int32),
            pltpu.VMEM((n_pad,), jnp.float32),
            pltpu.VMEM((_NS, nchunk), jnp.float32),
            pltpu.VMEM((nchunk,), jnp.float32),
            pltpu.VMEM_SHARED((_NS, n_pad), jnp.float32),
        ],
    )
    def body(ei_hbm, zeros_hbm, out_hbm, idx_v, ind_v, blk_v, acc_v, shared):
        cid = lax.axis_index("c")
        sid = lax.axis_index("s")
        wid = sid * nc + cid
        ones = jnp.ones((_L,), jnp.float32)

        start = jnp.minimum(wid * chunk, E - chunk)

        pltpu.sync_copy(zeros_hbm, ind_v)
        pltpu.sync_copy(ei_hbm.at[:, pl.ds(start, chunk)], idx_v)

        def scat_body(g, carry):
            idx = idx_v[0, pl.ds(pl.multiple_of(g * _L, _L), _L)]
            plsc.store_scatter(ind_v, [idx], ones)
            return carry

        lax.fori_loop(0, chunk // _L, scat_body, 0, unroll=8)

        pltpu.sync_copy(ind_v, shared.at[sid])
        plsc.subcore_barrier()
        cs = sid * nchunk
        pltpu.sync_copy(shared.at[:, pl.ds(cs, nchunk)], blk_v)

        def merge_body(g, carry):
            o = pl.multiple_of(g * _L, _L)
            acc = blk_v[0, pl.ds(o, _L)]
            for t in range(1, _NS):
                acc = jnp.maximum(acc, blk_v[t, pl.ds(o, _L)])
            acc_v[pl.ds(o, _L)] = acc
            return carry

        lax.fori_loop(0, nchunk // _L, merge_body, 0, unroll=2)

        pltpu.sync_copy(acc_v, out_hbm.at[cid, pl.ds(cs, nchunk)])

    return body


_INV_SQRT2 = 0.7071067811865476


def _tc_dense_body(x_ref, w_ref, w1_ref, b1_ref, w2_ref, b2_ref, h_ref,
                   pw_ref):
    h = jnp.dot(x_ref[...], w_ref[...], preferred_element_type=jnp.float32)
    t = jnp.dot(h, w1_ref[...], preferred_element_type=jnp.float32)
    t = t + b1_ref[...]
    g = 0.5 * t * (1.0 + lax.erf(t * _INV_SQRT2))
    p = jnp.sum(g * w2_ref[...], axis=1, keepdims=True) + b2_ref[...]
    h_ref[...] = h
    pw_ref[...] = 1.0 / (1.0 + jnp.exp(-p))


@functools.lru_cache(maxsize=None)
def _tc_dense(N: int, IN: int, OUT: int, HID: int, R: int):
    assert N % R == 0
    grid = (N // R,)
    return pl.pallas_call(
        _tc_dense_body,
        grid=grid,
        in_specs=[
            pl.BlockSpec((R, IN), lambda i: (i, 0)),
            pl.BlockSpec((IN, OUT), lambda i: (0, 0)),
            pl.BlockSpec((OUT, HID), lambda i: (0, 0)),
            pl.BlockSpec((1, HID), lambda i: (0, 0)),
            pl.BlockSpec((1, HID), lambda i: (0, 0)),
            pl.BlockSpec((1, 1), lambda i: (0, 0)),
        ],
        out_specs=[
            pl.BlockSpec((R, OUT), lambda i: (i, 0)),
            pl.BlockSpec((R, 1), lambda i: (i, 0)),
        ],
        out_shape=[
            jax.ShapeDtypeStruct((N, OUT), jnp.float32),
            jax.ShapeDtypeStruct((N, 1), jnp.float32),
        ],
        compiler_params=pltpu.CompilerParams(
            dimension_semantics=("parallel",)),
    )


def _tc_scale_body(h_ref, pw_ref, bias_ref, ind_ref, out_ref):
    ind = jnp.max(ind_ref[...], axis=0, keepdims=True)
    indT = jnp.transpose(ind)
    scale = jnp.where(indT > 0.0, 1.0 + pw_ref[...], 0.0)
    out_ref[...] = h_ref[...] * scale + bias_ref[...]


@functools.lru_cache(maxsize=None)
def _tc_scale(N: int, OUT: int, nc: int, R: int):
    grid = ((N + R - 1) // R,)
    return pl.pallas_call(
        _tc_scale_body,
        grid=grid,
        in_specs=[
            pl.BlockSpec((R, OUT), lambda i: (i, 0)),
            pl.BlockSpec((R, 1), lambda i: (i, 0)),
            pl.BlockSpec((1, OUT), lambda i: (0, 0)),
            pl.BlockSpec((nc, R), lambda i: (0, i)),
        ],
        out_specs=pl.BlockSpec((R, OUT), lambda i: (i, 0)),
        out_shape=jax.ShapeDtypeStruct((N, OUT), jnp.float32),
        compiler_params=pltpu.CompilerParams(
            dimension_semantics=("parallel",)),
    )


def kernel(x, edge_index, W, att, bias, pd_w1, pd_b1, pd_w2, pd_b2):
    del att
    N, IN = x.shape
    OUT = W.shape[1]
    HID = pd_w1.shape[1]
    E = edge_index.shape[1]

    n_pad = _NS * (((N + _NS - 1) // _NS + 127) // 128 * 128)
    ind2 = _sc_indicator(E, N)(edge_index, jnp.zeros((n_pad,), jnp.float32))

    h, pw = _tc_dense(N, IN, OUT, HID, 2000)(
        x, W, pd_w1,
        pd_b1.reshape(1, HID),
        pd_w2.reshape(1, HID),
        pd_b2.reshape(1, 1),
    )
    return _tc_scale(N, OUT, _NC, 2048)(h, pw, bias.reshape(1, OUT), ind2)


# speedup vs baseline: 1.2552x; 1.0222x over previous
import functools

import jax
import jax.numpy as jnp
from jax import lax
from jax.experimental import pallas as pl
from jax.experimental.pallas import tpu as pltpu
from jax.experimental.pallas import tpu_sc as plsc

_NC = 2
_NS = 16
_NW = _NC * _NS
_L = 16


@functools.lru_cache(maxsize=None)
def _sc_indicator(E: int, N: int, nc: int = _NC):
    nw = nc * _NS
    assert E % 128 == 0 and N % _L == 0
    chunk = ((E + nw - 1) // nw + 127) // 128 * 128
    assert chunk % _L == 0
    mesh = plsc.VectorSubcoreMesh(core_axis_name="c", subcore_axis_name="s",
                                  num_cores=nc)

    nchunk = ((N + _NS - 1) // _NS + 127) // 128 * 128
    n_pad = _NS * nchunk

    @functools.partial(
        pl.kernel,
        mesh=mesh,
        compiler_params=pltpu.CompilerParams(needs_layout_passes=False),
        out_type=jax.ShapeDtypeStruct((nc, n_pad), jnp.float32),
        scratch_types=[
            pltpu.VMEM((2, chunk), jnp.int32),
            pltpu.VMEM((n_pad,), jnp.float32),
            pltpu.VMEM((_NS, nchunk), jnp.float32),
            pltpu.VMEM((nchunk,), jnp.float32),
            pltpu.VMEM_SHARED((_NS, n_pad), jnp.float32),
        ],
    )
    def body(ei_hbm, zeros_hbm, out_hbm, idx_v, ind_v, blk_v, acc_v, shared):
        cid = lax.axis_index("c")
        sid = lax.axis_index("s")
        wid = sid * nc + cid
        ones = jnp.ones((_L,), jnp.float32)

        start = jnp.minimum(wid * chunk, E - chunk)

        pltpu.sync_copy(zeros_hbm, ind_v)
        pltpu.sync_copy(ei_hbm.at[:, pl.ds(start, chunk)], idx_v)

        def scat_body(g, carry):
            idx = idx_v[0, pl.ds(pl.multiple_of(g * _L, _L), _L)]
            plsc.store_scatter(ind_v, [idx], ones)
            return carry

        lax.fori_loop(0, chunk // _L, scat_body, 0, unroll=8)

        pltpu.sync_copy(ind_v, shared.at[sid])
        plsc.subcore_barrier()
        cs = sid * nchunk
        pltpu.sync_copy(shared.at[:, pl.ds(cs, nchunk)], blk_v)

        def merge_body(g, carry):
            o = pl.multiple_of(g * _L, _L)
            acc = blk_v[0, pl.ds(o, _L)]
            for t in range(1, _NS):
                acc = jnp.maximum(acc, blk_v[t, pl.ds(o, _L)])
            acc_v[pl.ds(o, _L)] = acc
            return carry

        lax.fori_loop(0, nchunk // _L, merge_body, 0, unroll=2)

        pltpu.sync_copy(acc_v, out_hbm.at[cid, pl.ds(cs, nchunk)])

    return body


_INV_SQRT2 = 0.7071067811865476


def _tc_dense_body(x_ref, w_ref, w1_ref, b1_ref, w2_ref, b2_ref, pw_ref):
    h = jnp.dot(x_ref[...], w_ref[...], preferred_element_type=jnp.float32)
    t = jnp.dot(h, w1_ref[...], preferred_element_type=jnp.float32)
    t = t + b1_ref[...]
    g = 0.5 * t * (1.0 + lax.erf(t * _INV_SQRT2))
    p = jnp.sum(g * w2_ref[...], axis=1, keepdims=True) + b2_ref[...]
    pw_ref[...] = 1.0 / (1.0 + jnp.exp(-p))


@functools.lru_cache(maxsize=None)
def _tc_dense(N: int, IN: int, OUT: int, HID: int, R: int):
    assert N % R == 0
    grid = (N // R,)
    return pl.pallas_call(
        _tc_dense_body,
        grid=grid,
        in_specs=[
            pl.BlockSpec((R, IN), lambda i: (i, 0)),
            pl.BlockSpec((IN, OUT), lambda i: (0, 0)),
            pl.BlockSpec((OUT, HID), lambda i: (0, 0)),
            pl.BlockSpec((1, HID), lambda i: (0, 0)),
            pl.BlockSpec((1, HID), lambda i: (0, 0)),
            pl.BlockSpec((1, 1), lambda i: (0, 0)),
        ],
        out_specs=pl.BlockSpec((R, 1), lambda i: (i, 0)),
        out_shape=jax.ShapeDtypeStruct((N, 1), jnp.float32),
        compiler_params=pltpu.CompilerParams(
            dimension_semantics=("parallel",)),
    )


def _tc_scale_body(x_ref, w_ref, pw_ref, bias_ref, ind_ref, out_ref):
    h = jnp.dot(x_ref[...], w_ref[...], preferred_element_type=jnp.float32)
    ind = jnp.max(ind_ref[...], axis=0, keepdims=True)
    indT = jnp.transpose(ind)
    scale = jnp.where(indT > 0.0, 1.0 + pw_ref[...], 0.0)
    out_ref[...] = h * scale + bias_ref[...]


@functools.lru_cache(maxsize=None)
def _tc_scale(N: int, IN: int, OUT: int, nc: int, R: int):
    grid = ((N + R - 1) // R,)
    return pl.pallas_call(
        _tc_scale_body,
        grid=grid,
        in_specs=[
            pl.BlockSpec((R, IN), lambda i: (i, 0)),
            pl.BlockSpec((IN, OUT), lambda i: (0, 0)),
            pl.BlockSpec((R, 1), lambda i: (i, 0)),
            pl.BlockSpec((1, OUT), lambda i: (0, 0)),
            pl.BlockSpec((nc, R), lambda i: (0, i)),
        ],
        out_specs=pl.BlockSpec((R, OUT), lambda i: (i, 0)),
        out_shape=jax.ShapeDtypeStruct((N, OUT), jnp.float32),
        compiler_params=pltpu.CompilerParams(
            dimension_semantics=("parallel",)),
    )


def kernel(x, edge_index, W, att, bias, pd_w1, pd_b1, pd_w2, pd_b2):
    del att
    N, IN = x.shape
    OUT = W.shape[1]
    HID = pd_w1.shape[1]
    E = edge_index.shape[1]

    n_pad = _NS * (((N + _NS - 1) // _NS + 127) // 128 * 128)
    ind2 = _sc_indicator(E, N)(edge_index, jnp.zeros((n_pad,), jnp.float32))

    pw = _tc_dense(N, IN, OUT, HID, 2000)(
        x, W, pd_w1,
        pd_b1.reshape(1, HID),
        pd_w2.reshape(1, HID),
        pd_b2.reshape(1, 1),
    )
    return _tc_scale(N, IN, OUT, _NC, 2048)(x, W, pw, bias.reshape(1, OUT),
                                            ind2)
